# baseline (speedup 1.0000x reference)
import functools

import numpy as np
import jax
import jax.numpy as jnp
from jax import lax
from jax.experimental import pallas as pl
from jax.experimental.pallas import tpu as pltpu
from jax.experimental.pallas import tpu_sc as plsc

N_EMBED = 26
VOCAB = 100000
D_EMBED = 32
D_MODEL = 768
T = 512
B = 32
NTOK = T * B
NFEAT = N_EMBED * D_EMBED

NC = 2
NS = 16
NW = NC * NS
LINES_PER_W = NFEAT // NW

CH = 4096
NCH = NTOK // CH


def _pos_encoding(d_model, max_len):
    position = np.arange(max_len, dtype=np.float32)[:, None]
    div_term = np.exp(
        np.arange(0, d_model, 2, dtype=np.float32) * (-np.log(10000.0) / d_model)
    )
    pe = np.zeros((max_len, d_model), dtype=np.float32)
    pe[:, 0::2] = np.sin(position * div_term)
    pe[:, 1::2] = np.cos(position * div_term)
    return pe


_PE = _pos_encoding(D_MODEL, T)


def _gather_body(xf_hbm, tab_hbm, h_hbm, line_v, idx_v, out_v, sem):
    wid = lax.axis_index("s") * NC + lax.axis_index("c")

    def do_line(k, carry):
        f = wid * LINES_PER_W + k
        i = f >> 5
        d = f & 31

        @pl.when(jnp.logical_or(k == 0, d == 0))
        def _():
            pltpu.sync_copy(xf_hbm.at[pl.ds(i * NTOK, NTOK)], idx_v)

        pltpu.sync_copy(tab_hbm.at[i, d], line_v)

        def do_chunk(c, carry2):
            @plsc.parallel_loop(0, CH // 16, unroll=8)
            def do_vec(j):
                ids = idx_v[pl.ds(c * CH + j * 16, 16)]
                out_v[pl.ds(j * 16, 16)] = plsc.load_gather(line_v, [ids])
            pltpu.sync_copy(out_v, h_hbm.at[f, pl.ds(c * CH, CH)])
            return carry2

        lax.fori_loop(0, NCH, do_chunk, 0)
        return carry

    lax.fori_loop(0, LINES_PER_W, do_line, 0)


_gather = functools.partial(
    pl.kernel,
    mesh=plsc.VectorSubcoreMesh(core_axis_name="c", subcore_axis_name="s"),
    compiler_params=pltpu.CompilerParams(needs_layout_passes=False),
    out_type=jax.ShapeDtypeStruct((NFEAT, NTOK), jnp.float32),
    scratch_types=[
        pltpu.VMEM((VOCAB,), jnp.float32),
        pltpu.VMEM((NTOK,), jnp.int32),
        pltpu.VMEM((CH,), jnp.float32),
        pltpu.SemaphoreType.DMA,
    ],
)(_gather_body)


TM = 64


def _proj_body(h_ref, w_ref, b_ref, pe_ref, out_ref):
    h_bf = h_ref[...].astype(jnp.bfloat16)
    acc = lax.dot_general(
        h_bf,
        w_ref[...],
        (((0,), (0,)), ((), ())),
        preferred_element_type=jnp.float32,
    )
    acc = acc + b_ref[...]
    acc = jnp.maximum(acc, 0.0)
    acc = acc.reshape(TM, B, D_MODEL) + pe_ref[...][:, None, :]
    out_ref[...] = acc


def _projection(h, w_bf, b2, pe):
    return pl.pallas_call(
        _proj_body,
        grid=(T // TM,),
        in_specs=[
            pl.BlockSpec((NFEAT, TM * B), lambda m: (0, m)),
            pl.BlockSpec((NFEAT, D_MODEL), lambda m: (0, 0)),
            pl.BlockSpec((1, D_MODEL), lambda m: (0, 0)),
            pl.BlockSpec((TM, D_MODEL), lambda m: (m, 0)),
        ],
        out_specs=pl.BlockSpec((TM, B, D_MODEL), lambda m: (m, 0, 0)),
        out_shape=jax.ShapeDtypeStruct((T, B, D_MODEL), jnp.float32),
    )(h, w_bf, b2, pe)


def kernel(x, tables, W, b):
    tab_t = jnp.transpose(tables, (0, 2, 1))
    xf = x.reshape(-1)
    h = _gather(xf, tab_t)
    return _projection(
        h,
        W.astype(jnp.bfloat16),
        b.reshape(1, D_MODEL),
        _PE,
    )


# speedup vs baseline: 1.0803x; 1.0803x over previous
import functools

import numpy as np
import jax
import jax.numpy as jnp
from jax import lax
from jax.experimental import pallas as pl
from jax.experimental.pallas import tpu as pltpu
from jax.experimental.pallas import tpu_sc as plsc

N_EMBED = 26
VOCAB = 100000
D_EMBED = 32
D_MODEL = 768
T = 512
B = 32
NTOK = T * B
NFEAT = N_EMBED * D_EMBED

NC = 2
NS = 16
NW = NC * NS
LINES_PER_W = NFEAT // NW

CH = 4096
NCH = NTOK // CH


def _pos_encoding(d_model, max_len):
    position = np.arange(max_len, dtype=np.float32)[:, None]
    div_term = np.exp(
        np.arange(0, d_model, 2, dtype=np.float32) * (-np.log(10000.0) / d_model)
    )
    pe = np.zeros((max_len, d_model), dtype=np.float32)
    pe[:, 0::2] = np.sin(position * div_term)
    pe[:, 1::2] = np.cos(position * div_term)
    return pe


_PE = _pos_encoding(D_MODEL, T)


NSLOT = 3


def _gather_body(xf_hbm, tab_hbm, h_hbm, line_v, idx_v, out_v, s0, s1, s2):
    wid = lax.axis_index("s") * NC + lax.axis_index("c")
    sems = [s0, s1, s2]

    def out_desc(f, c, slot):
        return pltpu.make_async_copy(
            out_v.at[pl.ds(slot * CH, CH)],
            h_hbm.at[f, pl.ds(c * CH, CH)],
            sems[slot],
        )

    def do_line(k, carry):
        f = wid * LINES_PER_W + k
        i = f >> 5
        d = f & 31

        @pl.when(jnp.logical_or(k == 0, d == 0))
        def _():
            pltpu.sync_copy(xf_hbm.at[pl.ds(i * NTOK, NTOK)], idx_v)

        pltpu.sync_copy(tab_hbm.at[i, d], line_v)

        for c in range(NCH):
            slot = c % NSLOT
            if c >= NSLOT:
                out_desc(f, c - NSLOT, slot).wait()
            else:
                @pl.when(k > 0)
                def _(slot=slot, c=c):
                    out_desc(f, c, slot).wait()

            @plsc.parallel_loop(0, CH // 16, unroll=8)
            def do_vec(j, c=c, slot=slot):
                ids = idx_v[pl.ds(c * CH + j * 16, 16)]
                out_v[pl.ds(slot * CH + j * 16, 16)] = plsc.load_gather(
                    line_v, [ids]
                )

            out_desc(f, c, slot).start()
        return carry

    lax.fori_loop(0, LINES_PER_W, do_line, 0)
    for slot in range(NSLOT):
        out_desc(wid * LINES_PER_W, 0, slot).wait()


_gather = functools.partial(
    pl.kernel,
    mesh=plsc.VectorSubcoreMesh(core_axis_name="c", subcore_axis_name="s"),
    compiler_params=pltpu.CompilerParams(needs_layout_passes=False),
    out_type=jax.ShapeDtypeStruct((NFEAT, NTOK), jnp.float32),
    scratch_types=[
        pltpu.VMEM((VOCAB,), jnp.float32),
        pltpu.VMEM((NTOK,), jnp.int32),
        pltpu.VMEM((NSLOT * CH,), jnp.float32),
        pltpu.SemaphoreType.DMA,
        pltpu.SemaphoreType.DMA,
        pltpu.SemaphoreType.DMA,
    ],
)(_gather_body)


TM = 64


def _proj_body(h_ref, w_ref, b_ref, pe_ref, out_ref):
    h_bf = h_ref[...].astype(jnp.bfloat16)
    acc = lax.dot_general(
        h_bf,
        w_ref[...],
        (((0,), (0,)), ((), ())),
        preferred_element_type=jnp.float32,
    )
    acc = acc + b_ref[...]
    acc = jnp.maximum(acc, 0.0)
    acc = acc.reshape(TM, B, D_MODEL) + pe_ref[...][:, None, :]
    out_ref[...] = acc


def _projection(h, w_bf, b2, pe):
    return pl.pallas_call(
        _proj_body,
        grid=(T // TM,),
        in_specs=[
            pl.BlockSpec((NFEAT, TM * B), lambda m: (0, m)),
            pl.BlockSpec((NFEAT, D_MODEL), lambda m: (0, 0)),
            pl.BlockSpec((1, D_MODEL), lambda m: (0, 0)),
            pl.BlockSpec((TM, D_MODEL), lambda m: (m, 0)),
        ],
        out_specs=pl.BlockSpec((TM, B, D_MODEL), lambda m: (m, 0, 0)),
        out_shape=jax.ShapeDtypeStruct((T, B, D_MODEL), jnp.float32),
    )(h, w_bf, b2, pe)


def kernel(x, tables, W, b):
    tab_t = jnp.transpose(tables, (0, 2, 1))
    xf = x.reshape(-1)
    h = _gather(xf, tab_t)
    return _projection(
        h,
        W.astype(jnp.bfloat16),
        b.reshape(1, D_MODEL),
        _PE,
    )


# speedup vs baseline: 1.0814x; 1.0010x over previous
import functools

import numpy as np
import jax
import jax.numpy as jnp
from jax import lax
from jax.experimental import pallas as pl
from jax.experimental.pallas import tpu as pltpu
from jax.experimental.pallas import tpu_sc as plsc

N_EMBED = 26
VOCAB = 100000
D_EMBED = 32
D_MODEL = 768
T = 512
B = 32
NTOK = T * B
NFEAT = N_EMBED * D_EMBED

NC = 2
NS = 16
NW = NC * NS
LINES_PER_W = NFEAT // NW

CH = 4096
NCH = NTOK // CH


def _pos_encoding(d_model, max_len):
    position = np.arange(max_len, dtype=np.float32)[:, None]
    div_term = np.exp(
        np.arange(0, d_model, 2, dtype=np.float32) * (-np.log(10000.0) / d_model)
    )
    pe = np.zeros((max_len, d_model), dtype=np.float32)
    pe[:, 0::2] = np.sin(position * div_term)
    pe[:, 1::2] = np.cos(position * div_term)
    return pe


_PE = _pos_encoding(D_MODEL, T)


NSLOT = 3


def _gather_body(xf_hbm, tab_hbm, h_hbm, line_v, idx_v, out_v, s0, s1, s2):
    wid = lax.axis_index("s") * NC + lax.axis_index("c")
    sems = [s0, s1, s2]

    def out_desc(f, c, slot):
        return pltpu.make_async_copy(
            out_v.at[pl.ds(slot * CH, CH)],
            h_hbm.at[f, pl.ds(c * CH, CH)],
            sems[slot],
        )

    def do_line(k, carry):
        f = wid * LINES_PER_W + k
        i = f >> 5
        d = f & 31

        @pl.when(jnp.logical_or(k == 0, d == 0))
        def _():
            pltpu.sync_copy(xf_hbm.at[pl.ds(i * NTOK, NTOK)], idx_v)

        pltpu.sync_copy(tab_hbm.at[i, d], line_v)

        for c in range(NCH):
            slot = c % NSLOT
            if c >= NSLOT:
                out_desc(f, c - NSLOT, slot).wait()
            else:
                @pl.when(k > 0)
                def _(slot=slot, c=c):
                    out_desc(f, c, slot).wait()

            @plsc.parallel_loop(0, CH // 16, unroll=16)
            def do_vec(j, c=c, slot=slot):
                ids = idx_v[pl.ds(c * CH + j * 16, 16)]
                out_v[pl.ds(slot * CH + j * 16, 16)] = plsc.load_gather(
                    line_v, [ids]
                )

            out_desc(f, c, slot).start()
        return carry

    lax.fori_loop(0, LINES_PER_W, do_line, 0)
    for slot in range(NSLOT):
        out_desc(wid * LINES_PER_W, 0, slot).wait()


_gather = functools.partial(
    pl.kernel,
    mesh=plsc.VectorSubcoreMesh(core_axis_name="c", subcore_axis_name="s"),
    compiler_params=pltpu.CompilerParams(needs_layout_passes=False),
    out_type=jax.ShapeDtypeStruct((NFEAT, NTOK), jnp.float32),
    scratch_types=[
        pltpu.VMEM((VOCAB,), jnp.float32),
        pltpu.VMEM((NTOK,), jnp.int32),
        pltpu.VMEM((NSLOT * CH,), jnp.float32),
        pltpu.SemaphoreType.DMA,
        pltpu.SemaphoreType.DMA,
        pltpu.SemaphoreType.DMA,
    ],
)(_gather_body)


TM = 64


def _proj_body(h_ref, w_ref, b_ref, pe_ref, out_ref):
    h_bf = h_ref[...].astype(jnp.bfloat16)
    acc = lax.dot_general(
        h_bf,
        w_ref[...],
        (((0,), (0,)), ((), ())),
        preferred_element_type=jnp.float32,
    )
    acc = acc + b_ref[...]
    acc = jnp.maximum(acc, 0.0)
    acc = acc.reshape(TM, B, D_MODEL) + pe_ref[...][:, None, :]
    out_ref[...] = acc


def _projection(h, w_bf, b2, pe):
    return pl.pallas_call(
        _proj_body,
        grid=(T // TM,),
        in_specs=[
            pl.BlockSpec((NFEAT, TM * B), lambda m: (0, m)),
            pl.BlockSpec((NFEAT, D_MODEL), lambda m: (0, 0)),
            pl.BlockSpec((1, D_MODEL), lambda m: (0, 0)),
            pl.BlockSpec((TM, D_MODEL), lambda m: (m, 0)),
        ],
        out_specs=pl.BlockSpec((TM, B, D_MODEL), lambda m: (m, 0, 0)),
        out_shape=jax.ShapeDtypeStruct((T, B, D_MODEL), jnp.float32),
    )(h, w_bf, b2, pe)


def kernel(x, tables, W, b):
    tab_t = jnp.transpose(tables, (0, 2, 1))
    xf = x.reshape(-1)
    h = _gather(xf, tab_t)
    return _projection(
        h,
        W.astype(jnp.bfloat16),
        b.reshape(1, D_MODEL),
        _PE,
    )


# speedup vs baseline: 1.0850x; 1.0034x over previous
import functools

import numpy as np
import jax
import jax.numpy as jnp
from jax import lax
from jax.experimental import pallas as pl
from jax.experimental.pallas import tpu as pltpu
from jax.experimental.pallas import tpu_sc as plsc

N_EMBED = 26
VOCAB = 100000
D_EMBED = 32
D_MODEL = 768
T = 512
B = 32
NTOK = T * B
NFEAT = N_EMBED * D_EMBED

NC = 2
NS = 16
NW = NC * NS
LINES_PER_W = NFEAT // NW

CH = 4096
NCH = NTOK // CH


def _pos_encoding(d_model, max_len):
    position = np.arange(max_len, dtype=np.float32)[:, None]
    div_term = np.exp(
        np.arange(0, d_model, 2, dtype=np.float32) * (-np.log(10000.0) / d_model)
    )
    pe = np.zeros((max_len, d_model), dtype=np.float32)
    pe[:, 0::2] = np.sin(position * div_term)
    pe[:, 1::2] = np.cos(position * div_term)
    return pe


_PE = _pos_encoding(D_MODEL, T)


NSLOT = 3


def _gather_body(xf_hbm, tab_hbm, h_hbm, line_v, idx_v, out_v, s0, s1, s2):
    wid = lax.axis_index("s") * NC + lax.axis_index("c")
    sems = [s0, s1, s2]

    def out_desc(f, c, slot):
        return pltpu.make_async_copy(
            out_v.at[pl.ds(slot * CH, CH)],
            h_hbm.at[f, pl.ds(c * CH, CH)],
            sems[slot],
        )

    def do_line(k, carry):
        f = wid * LINES_PER_W + k
        i = f >> 5
        d = f & 31

        @pl.when(jnp.logical_or(k == 0, d == 0))
        def _():
            pltpu.sync_copy(xf_hbm.at[pl.ds(i * NTOK, NTOK)], idx_v)

        pltpu.sync_copy(tab_hbm.at[i, d], line_v)

        for c in range(NCH):
            slot = c % NSLOT
            if c >= NSLOT:
                out_desc(f, c - NSLOT, slot).wait()
            else:
                @pl.when(k > 0)
                def _(slot=slot, c=c):
                    out_desc(f, c, slot).wait()

            @plsc.parallel_loop(0, CH // 16, unroll=32)
            def do_vec(j, c=c, slot=slot):
                ids = idx_v[pl.ds(c * CH + j * 16, 16)]
                out_v[pl.ds(slot * CH + j * 16, 16)] = plsc.load_gather(
                    line_v, [ids]
                )

            out_desc(f, c, slot).start()
        return carry

    lax.fori_loop(0, LINES_PER_W, do_line, 0)
    for slot in range(NSLOT):
        out_desc(wid * LINES_PER_W, 0, slot).wait()


_gather = functools.partial(
    pl.kernel,
    mesh=plsc.VectorSubcoreMesh(core_axis_name="c", subcore_axis_name="s"),
    compiler_params=pltpu.CompilerParams(needs_layout_passes=False),
    out_type=jax.ShapeDtypeStruct((NFEAT, NTOK), jnp.float32),
    scratch_types=[
        pltpu.VMEM((VOCAB,), jnp.float32),
        pltpu.VMEM((NTOK,), jnp.int32),
        pltpu.VMEM((NSLOT * CH,), jnp.float32),
        pltpu.SemaphoreType.DMA,
        pltpu.SemaphoreType.DMA,
        pltpu.SemaphoreType.DMA,
    ],
)(_gather_body)


TM = 64


def _proj_body(h_ref, w_ref, b_ref, pe_ref, out_ref):
    h_bf = h_ref[...].astype(jnp.bfloat16)
    acc = lax.dot_general(
        h_bf,
        w_ref[...],
        (((0,), (0,)), ((), ())),
        preferred_element_type=jnp.float32,
    )
    acc = acc + b_ref[...]
    acc = jnp.maximum(acc, 0.0)
    acc = acc.reshape(TM, B, D_MODEL) + pe_ref[...][:, None, :]
    out_ref[...] = acc


def _projection(h, w_bf, b2, pe):
    return pl.pallas_call(
        _proj_body,
        grid=(T // TM,),
        in_specs=[
            pl.BlockSpec((NFEAT, TM * B), lambda m: (0, m)),
            pl.BlockSpec((NFEAT, D_MODEL), lambda m: (0, 0)),
            pl.BlockSpec((1, D_MODEL), lambda m: (0, 0)),
            pl.BlockSpec((TM, D_MODEL), lambda m: (m, 0)),
        ],
        out_specs=pl.BlockSpec((TM, B, D_MODEL), lambda m: (m, 0, 0)),
        out_shape=jax.ShapeDtypeStruct((T, B, D_MODEL), jnp.float32),
    )(h, w_bf, b2, pe)


def kernel(x, tables, W, b):
    tab_t = jnp.transpose(tables, (0, 2, 1))
    xf = x.reshape(-1)
    h = _gather(xf, tab_t)
    return _projection(
        h,
        W.astype(jnp.bfloat16),
        b.reshape(1, D_MODEL),
        _PE,
    )
